# initial kernel scaffold (unmeasured)
import jax
import jax.numpy as jnp
from jax import lax
from jax.experimental import pallas as pl
from jax.experimental.pallas import tpu as pltpu

N_DEV = 4
TM = 512


def _allreduce_relu(y):
    M, N = y.shape
    MB = M // N_DEV
    NH = N // 2

    def body(y_ref, out_ref, rbuf, va, vb, cp_sems,
             rs_send, rs_recv, ag_send, ag_recv):
        my = lax.axis_index("i")
        right = lax.rem(my + 1, N_DEV)
        left = lax.rem(my + N_DEV - 1, N_DEV)

        barrier = pltpu.get_barrier_semaphore()
        for nbr in (left, right):
            pl.semaphore_signal(barrier, inc=1, device_id=(nbr,),
                                device_id_type=pl.DeviceIdType.MESH)
        pl.semaphore_wait(barrier, 2)

        def send_blk(d, s):
            if d == 0:
                return lax.rem(my + (N_DEV - 1) - s, N_DEV)
            return lax.rem(my + 1 + s, N_DEV)

        def recv_blk(d, s):
            if d == 0:
                return lax.rem(my + (N_DEV - 2) - s + N_DEV, N_DEV)
            return lax.rem(my + 2 + s, N_DEV)

        def rs_add(d, s):
            rb = recv_blk(d, s)
            col0 = 0 if d == 0 else NH
            final = s == N_DEV - 2
            for i in range(MB // TM):
                ca = pltpu.make_async_copy(
                    rbuf.at[d, s, pl.ds(i * TM, TM), :], va, cp_sems.at[0])
                cb = pltpu.make_async_copy(
                    y_ref.at[pl.ds(rb * MB + i * TM, TM), pl.ds(col0, NH)],
                    vb, cp_sems.at[1])
                ca.start()
                cb.start()
                ca.wait()
                cb.wait()
                acc = va[...] + vb[...]
                if final:
                    acc = jnp.maximum(acc, 0.0)
                va[...] = acc
                if final:
                    co = pltpu.make_async_copy(
                        va,
                        out_ref.at[pl.ds(rb * MB + i * TM, TM),
                                   pl.ds(col0, NH)],
                        cp_sems.at[0])
                else:
                    co = pltpu.make_async_copy(
                        va, rbuf.at[d, s, pl.ds(i * TM, TM), :], cp_sems.at[0])
                co.start()
                co.wait()

        for s in range(N_DEV - 1):
            rdmas = []
            for d in range(2):
                tgt = right if d == 0 else left
                col0 = 0 if d == 0 else NH
                sb = send_blk(d, s)
                if s == 0:
                    src = y_ref.at[pl.ds(sb * MB, MB), pl.ds(col0, NH)]
                else:
                    src = rbuf.at[d, s - 1]
                rdma = pltpu.make_async_remote_copy(
                    src_ref=src,
                    dst_ref=rbuf.at[d, s],
                    send_sem=rs_send.at[d, s],
                    recv_sem=rs_recv.at[d, s],
                    device_id=(tgt,),
                    device_id_type=pl.DeviceIdType.MESH,
                )
                rdma.start()
                rdmas.append(rdma)
            for rdma in rdmas:
                rdma.wait()
            for d in range(2):
                rs_add(d, s)

        for t in range(N_DEV - 1):
            rdmas = []
            for d in range(2):
                tgt = right if d == 0 else left
                col0 = 0 if d == 0 else NH
                if d == 0:
                    sb = lax.rem(my + N_DEV - t, N_DEV)
                else:
                    sb = lax.rem(my + t, N_DEV)
                sl_rows = pl.ds(sb * MB, MB)
                sl_cols = pl.ds(col0, NH)
                rdma = pltpu.make_async_remote_copy(
                    src_ref=out_ref.at[sl_rows, sl_cols],
                    dst_ref=out_ref.at[sl_rows, sl_cols],
                    send_sem=ag_send.at[d, t],
                    recv_sem=ag_recv.at[d, t],
                    device_id=(tgt,),
                    device_id_type=pl.DeviceIdType.MESH,
                )
                rdma.start()
                rdmas.append(rdma)
            for rdma in rdmas:
                rdma.wait()

    return pl.pallas_call(
        body,
        out_shape=jax.ShapeDtypeStruct((M, N), jnp.float32),
        in_specs=[pl.BlockSpec(memory_space=pl.MemorySpace.ANY)],
        out_specs=pl.BlockSpec(memory_space=pl.MemorySpace.ANY),
        scratch_shapes=[
            pltpu.MemorySpace.HBM((2, N_DEV - 1, MB, NH), jnp.float32),
            pltpu.VMEM((TM, NH), jnp.float32),
            pltpu.VMEM((TM, NH), jnp.float32),
            pltpu.SemaphoreType.DMA((2,)),
            pltpu.SemaphoreType.DMA((2, N_DEV - 1)),
            pltpu.SemaphoreType.DMA((2, N_DEV - 1)),
            pltpu.SemaphoreType.DMA((2, N_DEV - 1)),
            pltpu.SemaphoreType.DMA((2, N_DEV - 1)),
        ],
        compiler_params=pltpu.CompilerParams(collective_id=0),
    )(y)


def kernel(x, w_mat):
    y = jnp.dot(x, w_mat, preferred_element_type=jnp.float32)
    return _allreduce_relu(y)


# baseline (device time: 1513757 ns/iter reference)
import jax
import jax.numpy as jnp
from jax import lax
from jax.experimental import pallas as pl
from jax.experimental.pallas import tpu as pltpu

N_DEV = 4
TM = 512


def _allreduce_relu(y):
    M, N = y.shape
    MB = M // N_DEV
    NH = N // 2

    def body(y_ref, out_ref, rbuf, va, vb, cp_sems,
             rs_send, rs_recv, ag_send, ag_recv):
        my = lax.axis_index("i")
        right = lax.rem(my + 1, N_DEV)
        left = lax.rem(my + N_DEV - 1, N_DEV)

        barrier = pltpu.get_barrier_semaphore()
        for nbr in (left, right):
            pl.semaphore_signal(barrier, inc=1, device_id=(nbr,),
                                device_id_type=pl.DeviceIdType.MESH)
        pl.semaphore_wait(barrier, 2)

        def send_blk(d, s):
            if d == 0:
                return lax.rem(my + (N_DEV - 1) - s, N_DEV)
            return lax.rem(my + 1 + s, N_DEV)

        def recv_blk(d, s):
            if d == 0:
                return lax.rem(my + (N_DEV - 2) - s + N_DEV, N_DEV)
            return lax.rem(my + 2 + s, N_DEV)

        def rs_add(d, s):
            rb = recv_blk(d, s)
            col0 = 0 if d == 0 else NH
            final = s == N_DEV - 2
            for i in range(MB // TM):
                ca = pltpu.make_async_copy(
                    rbuf.at[d, s, pl.ds(i * TM, TM), :], va, cp_sems.at[0])
                cb = pltpu.make_async_copy(
                    y_ref.at[pl.ds(rb * MB + i * TM, TM), pl.ds(col0, NH)],
                    vb, cp_sems.at[1])
                ca.start()
                cb.start()
                ca.wait()
                cb.wait()
                acc = va[...] + vb[...]
                if final:
                    acc = jnp.maximum(acc, 0.0)
                va[...] = acc
                if final:
                    co = pltpu.make_async_copy(
                        va,
                        out_ref.at[pl.ds(rb * MB + i * TM, TM),
                                   pl.ds(col0, NH)],
                        cp_sems.at[0])
                else:
                    co = pltpu.make_async_copy(
                        va, rbuf.at[d, s, pl.ds(i * TM, TM), :], cp_sems.at[0])
                co.start()
                co.wait()

        for s in range(N_DEV - 1):
            rdmas = []
            for d in range(2):
                tgt = right if d == 0 else left
                col0 = 0 if d == 0 else NH
                sb = send_blk(d, s)
                if s == 0:
                    src = y_ref.at[pl.ds(sb * MB, MB), pl.ds(col0, NH)]
                else:
                    src = rbuf.at[d, s - 1]
                rdma = pltpu.make_async_remote_copy(
                    src_ref=src,
                    dst_ref=rbuf.at[d, s],
                    send_sem=rs_send.at[d, s],
                    recv_sem=rs_recv.at[d, s],
                    device_id=(tgt,),
                    device_id_type=pl.DeviceIdType.MESH,
                )
                rdma.start()
                rdmas.append(rdma)
            for rdma in rdmas:
                rdma.wait()
            for d in range(2):
                rs_add(d, s)

        for t in range(N_DEV - 1):
            rdmas = []
            for d in range(2):
                tgt = right if d == 0 else left
                col0 = 0 if d == 0 else NH
                if d == 0:
                    sb = lax.rem(my + N_DEV - t, N_DEV)
                else:
                    sb = lax.rem(my + t, N_DEV)
                sl_rows = pl.ds(sb * MB, MB)
                sl_cols = pl.ds(col0, NH)
                rdma = pltpu.make_async_remote_copy(
                    src_ref=out_ref.at[sl_rows, sl_cols],
                    dst_ref=out_ref.at[sl_rows, sl_cols],
                    send_sem=ag_send.at[d, t],
                    recv_sem=ag_recv.at[d, t],
                    device_id=(tgt,),
                    device_id_type=pl.DeviceIdType.MESH,
                )
                rdma.start()
                rdmas.append(rdma)
            for rdma in rdmas:
                rdma.wait()

    out, _ = pl.pallas_call(
        body,
        out_shape=[
            jax.ShapeDtypeStruct((M, N), jnp.float32),
            jax.ShapeDtypeStruct((2, N_DEV - 1, MB, NH), jnp.float32),
        ],
        in_specs=[pl.BlockSpec(memory_space=pl.MemorySpace.ANY)],
        out_specs=[pl.BlockSpec(memory_space=pl.MemorySpace.ANY),
                   pl.BlockSpec(memory_space=pl.MemorySpace.ANY)],
        scratch_shapes=[
            pltpu.VMEM((TM, NH), jnp.float32),
            pltpu.VMEM((TM, NH), jnp.float32),
            pltpu.SemaphoreType.DMA((2,)),
            pltpu.SemaphoreType.DMA((2, N_DEV - 1)),
            pltpu.SemaphoreType.DMA((2, N_DEV - 1)),
            pltpu.SemaphoreType.DMA((2, N_DEV - 1)),
            pltpu.SemaphoreType.DMA((2, N_DEV - 1)),
        ],
        compiler_params=pltpu.CompilerParams(collective_id=0),
    )(y)
    return out


def kernel(x, w_mat):
    y = jnp.dot(x, w_mat, preferred_element_type=jnp.float32)
    return _allreduce_relu(y)


# device time: 1362802 ns/iter; 1.1108x vs baseline; 1.1108x over previous
import jax
import jax.numpy as jnp
from jax import lax
from jax.experimental import pallas as pl
from jax.experimental.pallas import tpu as pltpu

N_DEV = 4
TM = 512


def _allreduce_relu(y):
    M, N = y.shape
    MB = M // N_DEV
    NH = N // 2
    C = MB // TM

    def body(y_ref, out_ref, rbuf, va, vb, cp_sems,
             rs_send, rs_recv, ag_send, ag_recv):
        my = lax.axis_index("i")
        right = lax.rem(my + 1, N_DEV)
        left = lax.rem(my + N_DEV - 1, N_DEV)

        barrier = pltpu.get_barrier_semaphore()
        for nbr in (left, right):
            pl.semaphore_signal(barrier, inc=1, device_id=(nbr,),
                                device_id_type=pl.DeviceIdType.MESH)
        pl.semaphore_wait(barrier, 2)

        def send_blk(d, s):
            if d == 0:
                return lax.rem(my + (N_DEV - 1) - s, N_DEV)
            return lax.rem(my + 1 + s, N_DEV)

        def recv_blk(d, s):
            if d == 0:
                return lax.rem(my + (N_DEV - 2) - s + N_DEV, N_DEV)
            return lax.rem(my + 2 + s, N_DEV)

        def rs_rdma(d, s, c, src):
            tgt = right if d == 0 else left
            return pltpu.make_async_remote_copy(
                src_ref=src,
                dst_ref=rbuf.at[d, s, pl.ds(c * TM, TM), :],
                send_sem=rs_send.at[d, s, c],
                recv_sem=rs_recv.at[d, s, c],
                device_id=(tgt,),
                device_id_type=pl.DeviceIdType.MESH,
            )

        def ag_rdma(d, t, c):
            tgt = right if d == 0 else left
            col0 = 0 if d == 0 else NH
            if d == 0:
                sb = lax.rem(my + N_DEV - t, N_DEV)
            else:
                sb = lax.rem(my + t, N_DEV)
            sl = out_ref.at[pl.ds(sb * MB + c * TM, TM), pl.ds(col0, NH)]
            return pltpu.make_async_remote_copy(
                src_ref=sl,
                dst_ref=sl,
                send_sem=ag_send.at[d, t, c],
                recv_sem=ag_recv.at[d, t, c],
                device_id=(tgt,),
                device_id_type=pl.DeviceIdType.MESH,
            )

        sends = []

        for c in range(C):
            for d in range(2):
                col0 = 0 if d == 0 else NH
                sb = send_blk(d, 0)
                src = y_ref.at[pl.ds(sb * MB + c * TM, TM), pl.ds(col0, NH)]
                rdma = rs_rdma(d, 0, c, src)
                rdma.start()
                sends.append(rdma)

        for s in range(1, N_DEV - 1):
            for c in range(C):
                for d in range(2):
                    col0 = 0 if d == 0 else NH
                    rb = recv_blk(d, s - 1)
                    prev = rs_rdma(
                        d, s - 1, c, rbuf.at[d, s - 1, pl.ds(c * TM, TM), :])
                    prev.wait_recv()
                    ca = pltpu.make_async_copy(
                        rbuf.at[d, s - 1, pl.ds(c * TM, TM), :],
                        va, cp_sems.at[0])
                    cb = pltpu.make_async_copy(
                        y_ref.at[pl.ds(rb * MB + c * TM, TM),
                                 pl.ds(col0, NH)],
                        vb, cp_sems.at[1])
                    ca.start()
                    cb.start()
                    ca.wait()
                    cb.wait()
                    va[...] = va[...] + vb[...]
                    co = pltpu.make_async_copy(
                        va, rbuf.at[d, s - 1, pl.ds(c * TM, TM), :],
                        cp_sems.at[0])
                    co.start()
                    co.wait()
                    rdma = rs_rdma(
                        d, s, c, rbuf.at[d, s - 1, pl.ds(c * TM, TM), :])
                    rdma.start()
                    sends.append(rdma)

        for c in range(C):
            for d in range(2):
                col0 = 0 if d == 0 else NH
                prev = rs_rdma(
                    d, N_DEV - 2, c,
                    rbuf.at[d, N_DEV - 2, pl.ds(c * TM, TM), :])
                prev.wait_recv()
                ca = pltpu.make_async_copy(
                    rbuf.at[d, N_DEV - 2, pl.ds(c * TM, TM), :],
                    va, cp_sems.at[0])
                cb = pltpu.make_async_copy(
                    y_ref.at[pl.ds(my * MB + c * TM, TM), pl.ds(col0, NH)],
                    vb, cp_sems.at[1])
                ca.start()
                cb.start()
                ca.wait()
                cb.wait()
                va[...] = jnp.maximum(va[...] + vb[...], 0.0)
                co = pltpu.make_async_copy(
                    va,
                    out_ref.at[pl.ds(my * MB + c * TM, TM), pl.ds(col0, NH)],
                    cp_sems.at[0])
                co.start()
                co.wait()
                rdma = ag_rdma(d, 0, c)
                rdma.start()
                sends.append(rdma)

        for t in range(1, N_DEV - 1):
            for c in range(C):
                for d in range(2):
                    prev = ag_rdma(d, t - 1, c)
                    prev.wait_recv()
                    rdma = ag_rdma(d, t, c)
                    rdma.start()
                    sends.append(rdma)

        for c in range(C):
            for d in range(2):
                ag_rdma(d, N_DEV - 2, c).wait_recv()
        for rdma in sends:
            rdma.wait_send()

    out, _ = pl.pallas_call(
        body,
        out_shape=[
            jax.ShapeDtypeStruct((M, N), jnp.float32),
            jax.ShapeDtypeStruct((2, N_DEV - 1, MB, NH), jnp.float32),
        ],
        in_specs=[pl.BlockSpec(memory_space=pl.MemorySpace.ANY)],
        out_specs=[pl.BlockSpec(memory_space=pl.MemorySpace.ANY),
                   pl.BlockSpec(memory_space=pl.MemorySpace.ANY)],
        scratch_shapes=[
            pltpu.VMEM((TM, NH), jnp.float32),
            pltpu.VMEM((TM, NH), jnp.float32),
            pltpu.SemaphoreType.DMA((2,)),
            pltpu.SemaphoreType.DMA((2, N_DEV - 1, MB // TM)),
            pltpu.SemaphoreType.DMA((2, N_DEV - 1, MB // TM)),
            pltpu.SemaphoreType.DMA((2, N_DEV - 1, MB // TM)),
            pltpu.SemaphoreType.DMA((2, N_DEV - 1, MB // TM)),
        ],
        compiler_params=pltpu.CompilerParams(collective_id=0),
    )(y)
    return out


def kernel(x, w_mat):
    y = jnp.dot(x, w_mat, preferred_element_type=jnp.float32)
    return _allreduce_relu(y)


# device time: 1199752 ns/iter; 1.2617x vs baseline; 1.1359x over previous
import jax
import jax.numpy as jnp
from jax import lax
from jax.experimental import pallas as pl
from jax.experimental.pallas import tpu as pltpu

N_DEV = 4
TM = 512


def kernel(x, w_mat):
    M, K = x.shape
    N = w_mat.shape[1]
    MB = M // N_DEV
    NH = N // 2
    C = MB // TM

    def body(x_ref, w_ref, out_ref, rbuf, stage, w_vmem, va, vb, vx,
             cp_sems, rs_send, rs_recv, ag_send, ag_recv):
        my = lax.axis_index("i")
        right = lax.rem(my + 1, N_DEV)
        left = lax.rem(my + N_DEV - 1, N_DEV)

        barrier = pltpu.get_barrier_semaphore()
        for nbr in (left, right):
            pl.semaphore_signal(barrier, inc=1, device_id=(nbr,),
                                device_id_type=pl.DeviceIdType.MESH)
        pl.semaphore_wait(barrier, 2)

        cw = pltpu.make_async_copy(w_ref, w_vmem, cp_sems.at[2])
        cw.start()
        cw.wait()

        def send_blk(d, s):
            if d == 0:
                return lax.rem(my + (N_DEV - 1) - s, N_DEV)
            return lax.rem(my + 1 + s, N_DEV)

        def recv_blk(d, s):
            if d == 0:
                return lax.rem(my + 2 * N_DEV - 2 - s, N_DEV)
            return lax.rem(my + 2 + s, N_DEV)

        def partial_chunk(blk, c, d):
            col0 = 0 if d == 0 else NH
            cx = pltpu.make_async_copy(
                x_ref.at[pl.ds(blk * MB + c * TM, TM), :], vx, cp_sems.at[2])
            cx.start()
            cx.wait()
            return jnp.dot(vx[...], w_vmem[:, col0:col0 + NH],
                           preferred_element_type=jnp.float32)

        def rs_rdma(d, s, c, src):
            tgt = right if d == 0 else left
            return pltpu.make_async_remote_copy(
                src_ref=src,
                dst_ref=rbuf.at[d, s, pl.ds(c * TM, TM), :],
                send_sem=rs_send.at[d, s, c],
                recv_sem=rs_recv.at[d, s, c],
                device_id=(tgt,),
                device_id_type=pl.DeviceIdType.MESH,
            )

        def ag_rdma(d, t, c):
            tgt = right if d == 0 else left
            col0 = 0 if d == 0 else NH
            if d == 0:
                sb = lax.rem(my + N_DEV - t, N_DEV)
            else:
                sb = lax.rem(my + t, N_DEV)
            sl = out_ref.at[pl.ds(sb * MB + c * TM, TM), pl.ds(col0, NH)]
            return pltpu.make_async_remote_copy(
                src_ref=sl,
                dst_ref=sl,
                send_sem=ag_send.at[d, t, c],
                recv_sem=ag_recv.at[d, t, c],
                device_id=(tgt,),
                device_id_type=pl.DeviceIdType.MESH,
            )

        def inject(c, _):
            for d in range(2):
                va[...] = partial_chunk(send_blk(d, 0), c, d)
                co = pltpu.make_async_copy(
                    va, stage.at[d, pl.ds(c * TM, TM), :], cp_sems.at[0])
                co.start()
                co.wait()
                rs_rdma(d, 0, c, stage.at[d, pl.ds(c * TM, TM), :]).start()
            return _

        lax.fori_loop(0, C, inject, None)

        def rs_mid(idx, _):
            s = 1 + idx // C
            c = lax.rem(idx, C)
            for d in range(2):
                chunk = rbuf.at[d, s - 1, pl.ds(c * TM, TM), :]
                rs_rdma(d, s - 1, c, chunk).wait_recv()
                cb = pltpu.make_async_copy(chunk, vb, cp_sems.at[1])
                cb.start()
                acc = partial_chunk(recv_blk(d, s - 1), c, d)
                cb.wait()
                va[...] = vb[...] + acc
                co = pltpu.make_async_copy(va, chunk, cp_sems.at[0])
                co.start()
                co.wait()
                rs_rdma(d, s, c, chunk).start()
            return _

        lax.fori_loop(0, (N_DEV - 2) * C, rs_mid, None)

        def finalize(c, _):
            for d in range(2):
                col0 = 0 if d == 0 else NH
                chunk = rbuf.at[d, N_DEV - 2, pl.ds(c * TM, TM), :]
                rs_rdma(d, N_DEV - 2, c, chunk).wait_recv()
                cb = pltpu.make_async_copy(chunk, vb, cp_sems.at[1])
                cb.start()
                acc = partial_chunk(my, c, d)
                cb.wait()
                va[...] = jnp.maximum(vb[...] + acc, 0.0)
                co = pltpu.make_async_copy(
                    va,
                    out_ref.at[pl.ds(my * MB + c * TM, TM), pl.ds(col0, NH)],
                    cp_sems.at[0])
                co.start()
                co.wait()
                ag_rdma(d, 0, c).start()
            return _

        lax.fori_loop(0, C, finalize, None)

        def ag_fwd(idx, _):
            t = 1 + idx // C
            c = lax.rem(idx, C)
            for d in range(2):
                ag_rdma(d, t - 1, c).wait_recv()
                ag_rdma(d, t, c).start()
            return _

        lax.fori_loop(0, (N_DEV - 2) * C, ag_fwd, None)

        def drain_all(idx, _):
            s = idx // C
            c = lax.rem(idx, C)
            for d in range(2):
                chunk = rbuf.at[d, s, pl.ds(c * TM, TM), :]
                rs_rdma(d, s, c, chunk).wait_send()
                ag_rdma(d, s, c).wait_send()
            return _

        def ag_last_recv(c, _):
            for d in range(2):
                ag_rdma(d, N_DEV - 2, c).wait_recv()
            return _

        lax.fori_loop(0, C, ag_last_recv, None)
        lax.fori_loop(0, (N_DEV - 1) * C, drain_all, None)

    out, _, _ = pl.pallas_call(
        body,
        out_shape=[
            jax.ShapeDtypeStruct((M, N), jnp.float32),
            jax.ShapeDtypeStruct((2, N_DEV - 1, MB, NH), jnp.float32),
            jax.ShapeDtypeStruct((2, MB, NH), jnp.float32),
        ],
        in_specs=[pl.BlockSpec(memory_space=pl.MemorySpace.ANY),
                  pl.BlockSpec(memory_space=pl.MemorySpace.ANY)],
        out_specs=[pl.BlockSpec(memory_space=pl.MemorySpace.ANY),
                   pl.BlockSpec(memory_space=pl.MemorySpace.ANY),
                   pl.BlockSpec(memory_space=pl.MemorySpace.ANY)],
        scratch_shapes=[
            pltpu.VMEM((K, N), jnp.float32),
            pltpu.VMEM((TM, NH), jnp.float32),
            pltpu.VMEM((TM, NH), jnp.float32),
            pltpu.VMEM((TM, K), jnp.float32),
            pltpu.SemaphoreType.DMA((3,)),
            pltpu.SemaphoreType.DMA((2, N_DEV - 1, MB // TM)),
            pltpu.SemaphoreType.DMA((2, N_DEV - 1, MB // TM)),
            pltpu.SemaphoreType.DMA((2, N_DEV - 1, MB // TM)),
            pltpu.SemaphoreType.DMA((2, N_DEV - 1, MB // TM)),
        ],
        compiler_params=pltpu.CompilerParams(
            collective_id=0,
            vmem_limit_bytes=56 * 1024 * 1024,
        ),
    )(x, w_mat)
    return out


# device time: 1199549 ns/iter; 1.2619x vs baseline; 1.0002x over previous
import jax
import jax.numpy as jnp
from jax import lax
from jax.experimental import pallas as pl
from jax.experimental.pallas import tpu as pltpu

N_DEV = 4
TM = 512


def kernel(x, w_mat):
    M, K = x.shape
    N = w_mat.shape[1]
    MB = M // N_DEV
    NH = N // 2
    C = MB // TM

    def body(x_ref, w_ref, out_ref, rbuf, stage, w_vmem, va, vb, vx,
             cp_sems, rs_send, rs_recv, ag_send, ag_recv):
        my = lax.axis_index("i")
        right = lax.rem(my + 1, N_DEV)
        left = lax.rem(my + N_DEV - 1, N_DEV)

        barrier = pltpu.get_barrier_semaphore()
        for nbr in (left, right):
            pl.semaphore_signal(barrier, inc=1, device_id=(nbr,),
                                device_id_type=pl.DeviceIdType.MESH)
        pl.semaphore_wait(barrier, 2)

        cw = pltpu.make_async_copy(w_ref, w_vmem, cp_sems.at[2])
        cw.start()
        cw.wait()

        def send_blk(d, s):
            if d == 0:
                return lax.rem(my + (N_DEV - 1) - s, N_DEV)
            return lax.rem(my + 1 + s, N_DEV)

        def recv_blk(d, s):
            if d == 0:
                return lax.rem(my + 2 * N_DEV - 2 - s, N_DEV)
            return lax.rem(my + 2 + s, N_DEV)

        def partial_chunk(blk, c, d):
            col0 = 0 if d == 0 else NH
            cx = pltpu.make_async_copy(
                x_ref.at[pl.ds(blk * MB + c * TM, TM), :], vx, cp_sems.at[2])
            cx.start()
            cx.wait()
            return jnp.dot(vx[...], w_vmem[:, col0:col0 + NH],
                           preferred_element_type=jnp.float32)

        def rs_rdma(d, s, c, src):
            tgt = right if d == 0 else left
            return pltpu.make_async_remote_copy(
                src_ref=src,
                dst_ref=rbuf.at[d, s, pl.ds(c * TM, TM), :],
                send_sem=rs_send.at[d, s, c],
                recv_sem=rs_recv.at[d, s, c],
                device_id=(tgt,),
                device_id_type=pl.DeviceIdType.MESH,
            )

        def ag_rdma(d, t, c):
            tgt = right if d == 0 else left
            col0 = 0 if d == 0 else NH
            if d == 0:
                sb = lax.rem(my + N_DEV - t, N_DEV)
            else:
                sb = lax.rem(my + t, N_DEV)
            sl = out_ref.at[pl.ds(sb * MB + c * TM, TM), pl.ds(col0, NH)]
            return pltpu.make_async_remote_copy(
                src_ref=sl,
                dst_ref=sl,
                send_sem=ag_send.at[d, t, c],
                recv_sem=ag_recv.at[d, t, c],
                device_id=(tgt,),
                device_id_type=pl.DeviceIdType.MESH,
            )

        def inject(c, _):
            for d in range(2):
                va[...] = partial_chunk(send_blk(d, 0), c, d)
                co = pltpu.make_async_copy(
                    va, stage.at[d, pl.ds(c * TM, TM), :], cp_sems.at[0])
                co.start()
                co.wait()
                rs_rdma(d, 0, c, stage.at[d, pl.ds(c * TM, TM), :]).start()
            return _

        lax.fori_loop(0, C, inject, None)

        def rs_mid(idx, _):
            s = 1 + idx // C
            c = lax.rem(idx, C)
            for d in range(2):
                chunk = rbuf.at[d, s - 1, pl.ds(c * TM, TM), :]
                acc = partial_chunk(recv_blk(d, s - 1), c, d)
                rs_rdma(d, s - 1, c, chunk).wait_recv()
                cb = pltpu.make_async_copy(chunk, vb, cp_sems.at[1])
                cb.start()
                cb.wait()
                va[...] = vb[...] + acc
                co = pltpu.make_async_copy(va, chunk, cp_sems.at[0])
                co.start()
                co.wait()
                rs_rdma(d, s, c, chunk).start()
            return _

        lax.fori_loop(0, (N_DEV - 2) * C, rs_mid, None)

        def finalize(c, _):
            for d in range(2):
                col0 = 0 if d == 0 else NH
                chunk = rbuf.at[d, N_DEV - 2, pl.ds(c * TM, TM), :]
                acc = partial_chunk(my, c, d)
                rs_rdma(d, N_DEV - 2, c, chunk).wait_recv()
                cb = pltpu.make_async_copy(chunk, vb, cp_sems.at[1])
                cb.start()
                cb.wait()
                va[...] = jnp.maximum(vb[...] + acc, 0.0)
                co = pltpu.make_async_copy(
                    va,
                    out_ref.at[pl.ds(my * MB + c * TM, TM), pl.ds(col0, NH)],
                    cp_sems.at[0])
                co.start()
                co.wait()
                ag_rdma(d, 0, c).start()
            return _

        lax.fori_loop(0, C, finalize, None)

        def ag_fwd(idx, _):
            t = 1 + idx // C
            c = lax.rem(idx, C)
            for d in range(2):
                ag_rdma(d, t - 1, c).wait_recv()
                ag_rdma(d, t, c).start()
            return _

        lax.fori_loop(0, (N_DEV - 2) * C, ag_fwd, None)

        def drain_all(idx, _):
            s = idx // C
            c = lax.rem(idx, C)
            for d in range(2):
                chunk = rbuf.at[d, s, pl.ds(c * TM, TM), :]
                rs_rdma(d, s, c, chunk).wait_send()
                ag_rdma(d, s, c).wait_send()
            return _

        def ag_last_recv(c, _):
            for d in range(2):
                ag_rdma(d, N_DEV - 2, c).wait_recv()
            return _

        lax.fori_loop(0, C, ag_last_recv, None)
        lax.fori_loop(0, (N_DEV - 1) * C, drain_all, None)

    out, _, _ = pl.pallas_call(
        body,
        out_shape=[
            jax.ShapeDtypeStruct((M, N), jnp.float32),
            jax.ShapeDtypeStruct((2, N_DEV - 1, MB, NH), jnp.float32),
            jax.ShapeDtypeStruct((2, MB, NH), jnp.float32),
        ],
        in_specs=[pl.BlockSpec(memory_space=pl.MemorySpace.ANY),
                  pl.BlockSpec(memory_space=pl.MemorySpace.ANY)],
        out_specs=[pl.BlockSpec(memory_space=pl.MemorySpace.ANY),
                   pl.BlockSpec(memory_space=pl.MemorySpace.ANY),
                   pl.BlockSpec(memory_space=pl.MemorySpace.ANY)],
        scratch_shapes=[
            pltpu.VMEM((K, N), jnp.float32),
            pltpu.VMEM((TM, NH), jnp.float32),
            pltpu.VMEM((TM, NH), jnp.float32),
            pltpu.VMEM((TM, K), jnp.float32),
            pltpu.SemaphoreType.DMA((3,)),
            pltpu.SemaphoreType.DMA((2, N_DEV - 1, MB // TM)),
            pltpu.SemaphoreType.DMA((2, N_DEV - 1, MB // TM)),
            pltpu.SemaphoreType.DMA((2, N_DEV - 1, MB // TM)),
            pltpu.SemaphoreType.DMA((2, N_DEV - 1, MB // TM)),
        ],
        compiler_params=pltpu.CompilerParams(
            collective_id=0,
            vmem_limit_bytes=56 * 1024 * 1024,
        ),
    )(x, w_mat)
    return out
